# Initial kernel scaffold; baseline (speedup 1.0000x reference)
import functools

import jax
import jax.numpy as jnp
from jax import lax
from jax.experimental import pallas as pl
from jax.experimental.pallas import tpu as pltpu
from jax.experimental.pallas import tpu_sc as plsc

NUM_USERS = 8192
NUM_ITEMS = 8192
N = NUM_USERS + NUM_ITEMS
NNZ = 268435
D = 64
BATCH = 4096

NC = 2
NS = 16
L = 16

CH = 128
E_PER_SUB = -(-NNZ // NS)
NCHUNK = -(-E_PER_SUB // CH)
E_SUB_PAD = NCHUNK * CH
E_PAD = E_SUB_PAD * NS

ROWS_PER_SUB = N // NS
BPW = BATCH // (NC * NS)

_mesh = plsc.VectorSubcoreMesh(core_axis_name="c", subcore_axis_name="s")


@functools.partial(
    pl.kernel,
    out_type=[jax.ShapeDtypeStruct((N, D), jnp.float32),
              jax.ShapeDtypeStruct((N, D), jnp.float32)],
    mesh=_mesh,
    scratch_types=[
        pltpu.VMEM((NCHUNK, CH), jnp.int32),
        pltpu.VMEM((NCHUNK, CH), jnp.int32),
        pltpu.VMEM((NCHUNK, CH), jnp.float32),
        pltpu.VMEM((CH, D), jnp.float32),
        pltpu.VMEM_SHARED((N, D), jnp.float32),
        pltpu.SemaphoreType.DMA,
    ],
)
def _layer(x0_hbm, x1_hbm, col_hbm, row_hbm, val_hbm, out0_hbm, out1_hbm,
           col_v, row_v, val_v, gbuf, acc_sh, sem):
    c = lax.axis_index("c")
    s = lax.axis_index("s")
    pltpu.sync_copy(col_hbm.at[c, s], col_v)
    pltpu.sync_copy(row_hbm.at[c, s], row_v)
    pltpu.sync_copy(val_hbm.at[c, s], val_v)

    def _zrow(e, carry):
        for q in range(D // L):
            gbuf[e, pl.ds(q * L, L)] = jnp.zeros((L,), jnp.float32)
        return carry
    lax.fori_loop(0, CH, _zrow, 0)
    for i in range(ROWS_PER_SUB // CH):
        pltpu.sync_copy(gbuf, acc_sh.at[pl.ds(s * ROWS_PER_SUB + i * CH, CH)])
    plsc.subcore_barrier()

    def _aspect(x_hbm):
        def _chunk(j, carry):
            pltpu.async_copy(x_hbm.at[col_v.at[j]], gbuf, sem).wait()

            def _scale(g, carry2):
                for t in range(L):
                    e = g * L + t
                    v = val_v[j, e]
                    for q in range(D // L):
                        gbuf[e, pl.ds(q * L, L)] = gbuf[e, pl.ds(q * L, L)] * v
                return carry2
            lax.fori_loop(0, CH // L, _scale, 0)
            pltpu.sync_copy(gbuf, acc_sh.at[row_v.at[j]], add=True)
            return carry
        lax.fori_loop(0, NCHUNK, _chunk, 0)

    @pl.when(c == 0)
    def _():
        _aspect(x0_hbm)

    @pl.when(c == 1)
    def _():
        _aspect(x1_hbm)

    plsc.subcore_barrier()

    @pl.when(c == 0)
    def _():
        pltpu.sync_copy(acc_sh.at[pl.ds(s * ROWS_PER_SUB, ROWS_PER_SUB)],
                        out0_hbm.at[pl.ds(s * ROWS_PER_SUB, ROWS_PER_SUB)])

    @pl.when(c == 1)
    def _():
        pltpu.sync_copy(acc_sh.at[pl.ds(s * ROWS_PER_SUB, ROWS_PER_SUB)],
                        out1_hbm.at[pl.ds(s * ROWS_PER_SUB, ROWS_PER_SUB)])


@functools.partial(
    pl.kernel,
    out_type=jax.ShapeDtypeStruct((BATCH,), jnp.float32),
    mesh=_mesh,
    scratch_types=[
        pltpu.VMEM((BPW,), jnp.int32),
        pltpu.VMEM((BPW,), jnp.int32),
        pltpu.VMEM((BPW,), jnp.int32),
        pltpu.VMEM((NUM_ITEMS,), jnp.int32),
        pltpu.VMEM((BPW, D), jnp.float32),
        pltpu.VMEM((BPW, D), jnp.float32),
        pltpu.VMEM((BPW, D), jnp.float32),
        pltpu.VMEM((BPW,), jnp.float32),
        pltpu.SemaphoreType.DMA,
    ],
)
def _readout(users_hbm, items_hbm, map_hbm,
             t00, t01, t02, t03, t10, t11, t12, t13, gamma_hbm,
             uidx, iidx, midx, map_v, uacc, iacc, tmp, gout, sem):
    c = lax.axis_index("c")
    s = lax.axis_index("s")
    wid = s * NC + c
    base = wid * BPW
    pltpu.sync_copy(users_hbm.at[pl.ds(base, BPW)], uidx)
    pltpu.sync_copy(items_hbm.at[pl.ds(base, BPW)], iidx)
    pltpu.sync_copy(map_hbm, map_v)

    for g in range(BPW // L):
        ivec = iidx[pl.ds(g * L, L)]
        m = plsc.load_gather(map_v, [ivec])
        midx[pl.ds(g * L, L)] = m + NUM_USERS
        iidx[pl.ds(g * L, L)] = ivec + NUM_USERS

    def _acc_into(accbuf, first_tbl, rest):
        pltpu.async_copy(first_tbl, accbuf, sem).wait()
        for tbl in rest:
            pltpu.async_copy(tbl, tmp, sem).wait()

            def _addrow(e, carry):
                for q in range(D // L):
                    sl = pl.ds(q * L, L)
                    accbuf[e, sl] = accbuf[e, sl] + tmp[e, sl]
                return carry
            lax.fori_loop(0, BPW, _addrow, 0)

    _acc_into(uacc, t00.at[uidx],
              [t.at[uidx] for t in (t01, t02, t03, t10, t11, t12, t13)])
    _acc_into(iacc, t00.at[iidx],
              [t.at[iidx] for t in (t01, t02, t03)] +
              [t.at[midx] for t in (t10, t11, t12, t13)])

    def _dot(e, carry):
        p = uacc[e, pl.ds(0, L)] * iacc[e, pl.ds(0, L)]
        for q in range(1, D // L):
            sl = pl.ds(q * L, L)
            p = p + uacc[e, sl] * iacc[e, sl]
        gout[e] = jnp.sum(p) * (1.0 / 64.0)
        return carry
    lax.fori_loop(0, BPW, _dot, 0)
    pltpu.sync_copy(gout, gamma_hbm.at[pl.ds(base, BPW)])


def _pad_edges(a):
    return jnp.pad(a, (0, E_PAD - NNZ)).reshape(NS, NCHUNK, CH)


def kernel(users, items, user_emb_0, user_emb_1, item_emb_0, item_emb_1,
           edge_row_0, edge_col_0, edge_val_0,
           edge_row_1, edge_col_1, edge_val_1, map_list):
    x00 = jnp.concatenate([user_emb_0, item_emb_0], axis=0)
    x10 = jnp.concatenate([user_emb_1, item_emb_1], axis=0)
    colp = jnp.stack([_pad_edges(edge_col_0), _pad_edges(edge_col_1)])
    rowp = jnp.stack([_pad_edges(edge_row_0), _pad_edges(edge_row_1)])
    valp = jnp.stack([_pad_edges(edge_val_0), _pad_edges(edge_val_1)])
    x01, x11 = _layer(x00, x10, colp, rowp, valp)
    x02, x12 = _layer(x01, x11, colp, rowp, valp)
    x03, x13 = _layer(x02, x12, colp, rowp, valp)
    return _readout(users, items, map_list,
                    x00, x01, x02, x03, x10, x11, x12, x13)


# speedup vs baseline: 4.9805x; 4.9805x over previous
import functools

import jax
import jax.numpy as jnp
from jax import lax
from jax.experimental import pallas as pl
from jax.experimental.pallas import tpu as pltpu
from jax.experimental.pallas import tpu_sc as plsc

NUM_USERS = 8192
NUM_ITEMS = 8192
N = NUM_USERS + NUM_ITEMS
NNZ = 268435
D = 64
BATCH = 4096

NC = 2
NS = 16
L = 16

CH = 128
E_PER_SUB = -(-NNZ // NS)
NCHUNK = -(-E_PER_SUB // CH)
E_SUB_PAD = NCHUNK * CH
E_PAD = E_SUB_PAD * NS

ROWS_PER_SUB = N // NS
BPW = BATCH // (NC * NS)

_mesh = plsc.VectorSubcoreMesh(core_axis_name="c", subcore_axis_name="s")
_params = pltpu.CompilerParams(use_tc_tiling_on_sc=False,
                               needs_layout_passes=False)


@functools.partial(
    pl.kernel,
    out_type=[jax.ShapeDtypeStruct((N, D), jnp.float32),
              jax.ShapeDtypeStruct((N, D), jnp.float32)],
    mesh=_mesh,
    scratch_types=[
        pltpu.VMEM((NCHUNK, CH), jnp.int32),
        pltpu.VMEM((NCHUNK, CH), jnp.int32),
        pltpu.VMEM((NCHUNK, CH), jnp.float32),
        pltpu.VMEM((CH, D), jnp.float32),
        pltpu.VMEM_SHARED((N, D), jnp.float32),
        pltpu.SemaphoreType.DMA,
    ],
    compiler_params=_params,
)
def _layer(x0_hbm, x1_hbm, col_hbm, row_hbm, val_hbm, out0_hbm, out1_hbm,
           col_v, row_v, val_v, gbuf, acc_sh, sem):
    c = lax.axis_index("c")
    s = lax.axis_index("s")
    pltpu.sync_copy(col_hbm.at[c, s], col_v)
    pltpu.sync_copy(row_hbm.at[c, s], row_v)
    pltpu.sync_copy(val_hbm.at[c, s], val_v)

    def _zrow(e, carry):
        for q in range(D // L):
            gbuf[e, pl.ds(q * L, L)] = jnp.zeros((L,), jnp.float32)
        return carry
    lax.fori_loop(0, CH, _zrow, 0)
    for i in range(ROWS_PER_SUB // CH):
        pltpu.sync_copy(gbuf, acc_sh.at[pl.ds(s * ROWS_PER_SUB + i * CH, CH)])
    plsc.subcore_barrier()

    def _aspect(x_hbm):
        def _chunk(j, carry):
            pltpu.async_copy(x_hbm.at[col_v.at[j]], gbuf, sem).wait()

            def _scale(g, carry2):
                vv = val_v[j, pl.ds(g * L, L)]
                for t in range(L):
                    e = g * L + t
                    v = vv[t]
                    for q in range(D // L):
                        gbuf[e, pl.ds(q * L, L)] = gbuf[e, pl.ds(q * L, L)] * v
                return carry2
            lax.fori_loop(0, CH // L, _scale, 0)
            pltpu.sync_copy(gbuf, acc_sh.at[row_v.at[j]], add=True)
            return carry
        lax.fori_loop(0, NCHUNK, _chunk, 0)

    @pl.when(c == 0)
    def _():
        _aspect(x0_hbm)

    @pl.when(c == 1)
    def _():
        _aspect(x1_hbm)

    plsc.subcore_barrier()

    @pl.when(c == 0)
    def _():
        pltpu.sync_copy(acc_sh.at[pl.ds(s * ROWS_PER_SUB, ROWS_PER_SUB)],
                        out0_hbm.at[pl.ds(s * ROWS_PER_SUB, ROWS_PER_SUB)])

    @pl.when(c == 1)
    def _():
        pltpu.sync_copy(acc_sh.at[pl.ds(s * ROWS_PER_SUB, ROWS_PER_SUB)],
                        out1_hbm.at[pl.ds(s * ROWS_PER_SUB, ROWS_PER_SUB)])


@functools.partial(
    pl.kernel,
    out_type=jax.ShapeDtypeStruct((BATCH,), jnp.float32),
    mesh=_mesh,
    scratch_types=[
        pltpu.VMEM((BPW,), jnp.int32),
        pltpu.VMEM((BPW,), jnp.int32),
        pltpu.VMEM((BPW,), jnp.int32),
        pltpu.VMEM((NUM_ITEMS,), jnp.int32),
        pltpu.VMEM((BPW, D), jnp.float32),
        pltpu.VMEM((BPW, D), jnp.float32),
        pltpu.VMEM((BPW, D), jnp.float32),
        pltpu.VMEM((BPW,), jnp.float32),
        pltpu.SemaphoreType.DMA,
    ],
    compiler_params=_params,
)
def _readout(users_hbm, items_hbm, map_hbm,
             t00, t01, t02, t03, t10, t11, t12, t13, gamma_hbm,
             uidx, iidx, midx, map_v, uacc, iacc, tmp, gout, sem):
    c = lax.axis_index("c")
    s = lax.axis_index("s")
    wid = s * NC + c
    base = wid * BPW
    pltpu.sync_copy(users_hbm.at[pl.ds(base, BPW)], uidx)
    pltpu.sync_copy(items_hbm.at[pl.ds(base, BPW)], iidx)
    pltpu.sync_copy(map_hbm, map_v)

    for g in range(BPW // L):
        ivec = iidx[pl.ds(g * L, L)]
        m = plsc.load_gather(map_v, [ivec])
        midx[pl.ds(g * L, L)] = m + NUM_USERS
        iidx[pl.ds(g * L, L)] = ivec + NUM_USERS

    def _acc_into(accbuf, first_tbl, rest):
        pltpu.async_copy(first_tbl, accbuf, sem).wait()
        for tbl in rest:
            pltpu.async_copy(tbl, tmp, sem).wait()

            def _addrow(e, carry):
                for q in range(D // L):
                    sl = pl.ds(q * L, L)
                    accbuf[e, sl] = accbuf[e, sl] + tmp[e, sl]
                return carry
            lax.fori_loop(0, BPW, _addrow, 0)

    _acc_into(uacc, t00.at[uidx],
              [t.at[uidx] for t in (t01, t02, t03, t10, t11, t12, t13)])
    _acc_into(iacc, t00.at[iidx],
              [t.at[iidx] for t in (t01, t02, t03)] +
              [t.at[midx] for t in (t10, t11, t12, t13)])

    lane = lax.broadcasted_iota(jnp.int32, (L,), 0)

    def _dot(g, carry):
        gvec = jnp.zeros((L,), jnp.float32)
        for t in range(L):
            e = g * L + t
            p = uacc[e, pl.ds(0, L)] * iacc[e, pl.ds(0, L)]
            for q in range(1, D // L):
                sl = pl.ds(q * L, L)
                p = p + uacc[e, sl] * iacc[e, sl]
            gvec = jnp.where(lane == t, jnp.sum(p) * (1.0 / 64.0), gvec)
        gout[pl.ds(g * L, L)] = gvec
        return carry
    lax.fori_loop(0, BPW // L, _dot, 0)
    pltpu.sync_copy(gout, gamma_hbm.at[pl.ds(base, BPW)])


def _pad_edges(a):
    return jnp.pad(a, (0, E_PAD - NNZ)).reshape(NS, NCHUNK, CH)


def kernel(users, items, user_emb_0, user_emb_1, item_emb_0, item_emb_1,
           edge_row_0, edge_col_0, edge_val_0,
           edge_row_1, edge_col_1, edge_val_1, map_list):
    x00 = jnp.concatenate([user_emb_0, item_emb_0], axis=0)
    x10 = jnp.concatenate([user_emb_1, item_emb_1], axis=0)
    colp = jnp.stack([_pad_edges(edge_col_0), _pad_edges(edge_col_1)])
    rowp = jnp.stack([_pad_edges(edge_row_0), _pad_edges(edge_row_1)])
    valp = jnp.stack([_pad_edges(edge_val_0), _pad_edges(edge_val_1)])
    x01, x11 = _layer(x00, x10, colp, rowp, valp)
    x02, x12 = _layer(x01, x11, colp, rowp, valp)
    x03, x13 = _layer(x02, x12, colp, rowp, valp)
    return _readout(users, items, map_list,
                    x00, x01, x02, x03, x10, x11, x12, x13)


# speedup vs baseline: 8.7327x; 1.7534x over previous
import functools

import jax
import jax.numpy as jnp
from jax import lax
from jax.experimental import pallas as pl
from jax.experimental.pallas import tpu as pltpu
from jax.experimental.pallas import tpu_sc as plsc

NUM_USERS = 8192
NUM_ITEMS = 8192
N = NUM_USERS + NUM_ITEMS
NNZ = 268435
D = 64
BATCH = 4096

NC = 2
NS = 16
L = 16

CH = 128
E_PER_SUB = -(-NNZ // NS)
NCHUNK = -(-E_PER_SUB // CH)
E_SUB_PAD = NCHUNK * CH
E_PAD = E_SUB_PAD * NS

ROWS_PER_SUB = N // NS
BPW = BATCH // (NC * NS)

NBUF = 4
NGRP = NCHUNK // NBUF
EB = 3
assert NCHUNK % NBUF == 0 and NGRP % EB == 0

_mesh = plsc.VectorSubcoreMesh(core_axis_name="c", subcore_axis_name="s")
_params = pltpu.CompilerParams(use_tc_tiling_on_sc=False,
                               needs_layout_passes=False)


@functools.partial(
    pl.kernel,
    out_type=[jax.ShapeDtypeStruct((N, D), jnp.float32),
              jax.ShapeDtypeStruct((N, D), jnp.float32)],
    mesh=_mesh,
    scratch_types=[
        pltpu.VMEM((EB, NBUF, CH), jnp.int32),
        pltpu.VMEM((EB, NBUF, CH), jnp.int32),
        pltpu.VMEM((EB, NBUF, CH), jnp.float32),
        pltpu.VMEM((NBUF, CH, D), jnp.float32),
        pltpu.VMEM_SHARED((N, D), jnp.float32),
        pltpu.SemaphoreType.DMA((EB,)),
        pltpu.SemaphoreType.DMA((NBUF,)),
        pltpu.SemaphoreType.DMA((NBUF,)),
    ],
    compiler_params=_params,
)
def _layer(x0_hbm, x1_hbm, col_hbm, row_hbm, val_hbm, out0_hbm, out1_hbm,
           colb, rowb, valb, gbuf, acc_sh, esem, gsem, ssem):
    c = lax.axis_index("c")
    s = lax.axis_index("s")

    def _eblock(g, u):
        pltpu.async_copy(col_hbm.at[c, s, pl.ds(g * NBUF, NBUF)],
                         colb.at[u], esem.at[u])
        pltpu.async_copy(row_hbm.at[c, s, pl.ds(g * NBUF, NBUF)],
                         rowb.at[u], esem.at[u])
        pltpu.async_copy(val_hbm.at[c, s, pl.ds(g * NBUF, NBUF)],
                         valb.at[u], esem.at[u])

    def _eblock_wait(g, u):
        pltpu.make_async_copy(col_hbm.at[c, s, pl.ds(g * NBUF, NBUF)],
                              colb.at[u], esem.at[u]).wait()
        pltpu.make_async_copy(row_hbm.at[c, s, pl.ds(g * NBUF, NBUF)],
                              rowb.at[u], esem.at[u]).wait()
        pltpu.make_async_copy(val_hbm.at[c, s, pl.ds(g * NBUF, NBUF)],
                              valb.at[u], esem.at[u]).wait()

    def _zrow(e, carry):
        for q in range(D // L):
            gbuf[0, e, pl.ds(q * L, L)] = jnp.zeros((L,), jnp.float32)
        return carry
    lax.fori_loop(0, CH, _zrow, 0)
    for i in range(ROWS_PER_SUB // CH):
        pltpu.async_copy(gbuf.at[0],
                         acc_sh.at[pl.ds(s * ROWS_PER_SUB + i * CH, CH)],
                         ssem.at[0])
    for i in range(ROWS_PER_SUB // CH):
        pltpu.make_async_copy(
            gbuf.at[0], acc_sh.at[pl.ds(s * ROWS_PER_SUB + i * CH, CH)],
            ssem.at[0]).wait()
    plsc.subcore_barrier()

    def _aspect(x_hbm):
        _eblock(0, 0)
        _eblock(1, 1)

        def _outer(gg, carry):
            for u in range(EB):
                g = gg * EB + u
                _eblock_wait(g, u)
                up = (u + EB - 1) % EB
                for b in range(NBUF):
                    if u == 0:
                        @pl.when(g > 0)
                        def _():
                            pltpu.make_async_copy(
                                gbuf.at[b], acc_sh.at[rowb.at[up, b]],
                                ssem.at[b]).wait()
                    else:
                        pltpu.make_async_copy(
                            gbuf.at[b], acc_sh.at[rowb.at[up, b]],
                            ssem.at[b]).wait()
                    pltpu.async_copy(x_hbm.at[colb.at[u, b]], gbuf.at[b],
                                     gsem.at[b])
                @pl.when(g + 2 < NGRP)
                def _():
                    _eblock(g + 2, (u + 2) % EB)
                for b in range(NBUF):
                    pltpu.make_async_copy(x_hbm.at[colb.at[u, b]],
                                          gbuf.at[b], gsem.at[b]).wait()

                    def _scale(hh, carry2):
                        vv = valb[u, b, pl.ds(hh * L, L)]
                        for t in range(L):
                            e = hh * L + t
                            v = vv[t]
                            for q in range(D // L):
                                gbuf[b, e, pl.ds(q * L, L)] = \
                                    gbuf[b, e, pl.ds(q * L, L)] * v
                        return carry2
                    lax.fori_loop(0, CH // L, _scale, 0)
                    pltpu.async_copy(gbuf.at[b], acc_sh.at[rowb.at[u, b]],
                                     ssem.at[b], add=True)
            return carry
        lax.fori_loop(0, NGRP // EB, _outer, 0)

        ul = (NGRP - 1) % EB
        for b in range(NBUF):
            pltpu.make_async_copy(gbuf.at[b], acc_sh.at[rowb.at[ul, b]],
                                  ssem.at[b]).wait()

    @pl.when(c == 0)
    def _():
        _aspect(x0_hbm)

    @pl.when(c == 1)
    def _():
        _aspect(x1_hbm)

    plsc.subcore_barrier()

    @pl.when(c == 0)
    def _():
        pltpu.sync_copy(acc_sh.at[pl.ds(s * ROWS_PER_SUB, ROWS_PER_SUB)],
                        out0_hbm.at[pl.ds(s * ROWS_PER_SUB, ROWS_PER_SUB)])

    @pl.when(c == 1)
    def _():
        pltpu.sync_copy(acc_sh.at[pl.ds(s * ROWS_PER_SUB, ROWS_PER_SUB)],
                        out1_hbm.at[pl.ds(s * ROWS_PER_SUB, ROWS_PER_SUB)])


@functools.partial(
    pl.kernel,
    out_type=jax.ShapeDtypeStruct((BATCH,), jnp.float32),
    mesh=_mesh,
    scratch_types=[
        pltpu.VMEM((BPW,), jnp.int32),
        pltpu.VMEM((BPW,), jnp.int32),
        pltpu.VMEM((BPW,), jnp.int32),
        pltpu.VMEM((NUM_ITEMS,), jnp.int32),
        pltpu.VMEM((BPW, D), jnp.float32),
        pltpu.VMEM((BPW, D), jnp.float32),
        pltpu.VMEM((BPW, D), jnp.float32),
        pltpu.VMEM((BPW,), jnp.float32),
        pltpu.SemaphoreType.DMA,
    ],
    compiler_params=_params,
)
def _readout(users_hbm, items_hbm, map_hbm,
             t00, t01, t02, t03, t10, t11, t12, t13, gamma_hbm,
             uidx, iidx, midx, map_v, uacc, iacc, tmp, gout, sem):
    c = lax.axis_index("c")
    s = lax.axis_index("s")
    wid = s * NC + c
    base = wid * BPW
    pltpu.sync_copy(users_hbm.at[pl.ds(base, BPW)], uidx)
    pltpu.sync_copy(items_hbm.at[pl.ds(base, BPW)], iidx)
    pltpu.sync_copy(map_hbm, map_v)

    for g in range(BPW // L):
        ivec = iidx[pl.ds(g * L, L)]
        m = plsc.load_gather(map_v, [ivec])
        midx[pl.ds(g * L, L)] = m + NUM_USERS
        iidx[pl.ds(g * L, L)] = ivec + NUM_USERS

    def _acc_into(accbuf, first_tbl, rest):
        pltpu.async_copy(first_tbl, accbuf, sem).wait()
        for tbl in rest:
            pltpu.async_copy(tbl, tmp, sem).wait()

            def _addrow(e, carry):
                for q in range(D // L):
                    sl = pl.ds(q * L, L)
                    accbuf[e, sl] = accbuf[e, sl] + tmp[e, sl]
                return carry
            lax.fori_loop(0, BPW, _addrow, 0)

    _acc_into(uacc, t00.at[uidx],
              [t.at[uidx] for t in (t01, t02, t03, t10, t11, t12, t13)])
    _acc_into(iacc, t00.at[iidx],
              [t.at[iidx] for t in (t01, t02, t03)] +
              [t.at[midx] for t in (t10, t11, t12, t13)])

    lane = lax.broadcasted_iota(jnp.int32, (L,), 0)

    def _dot(g, carry):
        gvec = jnp.zeros((L,), jnp.float32)
        for t in range(L):
            e = g * L + t
            p = uacc[e, pl.ds(0, L)] * iacc[e, pl.ds(0, L)]
            for q in range(1, D // L):
                sl = pl.ds(q * L, L)
                p = p + uacc[e, sl] * iacc[e, sl]
            gvec = jnp.where(lane == t, jnp.sum(p) * (1.0 / 64.0), gvec)
        gout[pl.ds(g * L, L)] = gvec
        return carry
    lax.fori_loop(0, BPW // L, _dot, 0)
    pltpu.sync_copy(gout, gamma_hbm.at[pl.ds(base, BPW)])


def _pad_edges(a):
    return jnp.pad(a, (0, E_PAD - NNZ)).reshape(NS, NCHUNK, CH)


def kernel(users, items, user_emb_0, user_emb_1, item_emb_0, item_emb_1,
           edge_row_0, edge_col_0, edge_val_0,
           edge_row_1, edge_col_1, edge_val_1, map_list):
    x00 = jnp.concatenate([user_emb_0, item_emb_0], axis=0)
    x10 = jnp.concatenate([user_emb_1, item_emb_1], axis=0)
    colp = jnp.stack([_pad_edges(edge_col_0), _pad_edges(edge_col_1)])
    rowp = jnp.stack([_pad_edges(edge_row_0), _pad_edges(edge_row_1)])
    valp = jnp.stack([_pad_edges(edge_val_0), _pad_edges(edge_val_1)])
    x01, x11 = _layer(x00, x10, colp, rowp, valp)
    x02, x12 = _layer(x01, x11, colp, rowp, valp)
    x03, x13 = _layer(x02, x12, colp, rowp, valp)
    return _readout(users, items, map_list,
                    x00, x01, x02, x03, x10, x11, x12, x13)


# speedup vs baseline: 15.3616x; 1.7591x over previous
import functools

import jax
import jax.numpy as jnp
from jax import lax
from jax.experimental import pallas as pl
from jax.experimental.pallas import tpu as pltpu
from jax.experimental.pallas import tpu_sc as plsc

NUM_USERS = 8192
NUM_ITEMS = 8192
N = NUM_USERS + NUM_ITEMS
NNZ = 268435
D = 64
BATCH = 4096
N_LAYERS = 3

NC = 2
NS = 16
L = 16

CH = 128
E_PER_SUB = -(-NNZ // NS)
NCHUNK = -(-E_PER_SUB // CH)
E_SUB_PAD = NCHUNK * CH
E_PAD = E_SUB_PAD * NS

ROWS_PER_SUB = N // NS
BPW = BATCH // (NC * NS)

NBUF = 4
NGRP = NCHUNK // NBUF
EB = 3
assert NCHUNK % NBUF == 0

_mesh = plsc.VectorSubcoreMesh(core_axis_name="c", subcore_axis_name="s")
_params = pltpu.CompilerParams(use_tc_tiling_on_sc=False,
                               needs_layout_passes=False)


@functools.partial(
    pl.kernel,
    out_type=jax.ShapeDtypeStruct((NC, N_LAYERS, N, D), jnp.float32),
    mesh=_mesh,
    scratch_types=[
        pltpu.VMEM((EB, NBUF, CH), jnp.int32),
        pltpu.VMEM((EB, NBUF, CH), jnp.int32),
        pltpu.VMEM((EB, NBUF, CH), jnp.float32),
        pltpu.VMEM((NBUF, CH, D), jnp.float32),
        pltpu.VMEM((2, CH, D), jnp.float32),
        pltpu.VMEM((CH, D), jnp.float32),
        pltpu.VMEM_SHARED((N, D), jnp.float32),
        pltpu.SemaphoreType.DMA((EB,)),
        pltpu.SemaphoreType.DMA((NBUF,)),
        pltpu.SemaphoreType.DMA((2,)),
        pltpu.SemaphoreType.DMA,
    ],
    compiler_params=_params,
)
def _propagate(x0s_hbm, col_hbm, row_hbm, val_hbm, xs_hbm,
               colb, rowb, valb, gbuf, sbuf, zbuf, acc_sh,
               esem, gsem, ssem, wsem):
    c = lax.axis_index("c")
    s = lax.axis_index("s")

    def _eb_issue(g, u):
        pltpu.async_copy(col_hbm.at[c, s, pl.ds(g * NBUF, NBUF)],
                         colb.at[u], esem.at[u])
        pltpu.async_copy(row_hbm.at[c, s, pl.ds(g * NBUF, NBUF)],
                         rowb.at[u], esem.at[u])
        pltpu.async_copy(val_hbm.at[c, s, pl.ds(g * NBUF, NBUF)],
                         valb.at[u], esem.at[u])

    def _eb_wait(g, u):
        pltpu.make_async_copy(col_hbm.at[c, s, pl.ds(g * NBUF, NBUF)],
                              colb.at[u], esem.at[u]).wait()
        pltpu.make_async_copy(row_hbm.at[c, s, pl.ds(g * NBUF, NBUF)],
                              rowb.at[u], esem.at[u]).wait()
        pltpu.make_async_copy(val_hbm.at[c, s, pl.ds(g * NBUF, NBUF)],
                              valb.at[u], esem.at[u]).wait()

    def _zero_acc():
        for i in range(ROWS_PER_SUB // CH):
            pltpu.async_copy(
                zbuf, acc_sh.at[pl.ds(s * ROWS_PER_SUB + i * CH, CH)], wsem)
        for i in range(ROWS_PER_SUB // CH):
            pltpu.make_async_copy(
                zbuf, acc_sh.at[pl.ds(s * ROWS_PER_SUB + i * CH, CH)],
                wsem).wait()

    def _zrow(e, carry):
        for q in range(D // L):
            zbuf[e, pl.ds(q * L, L)] = jnp.zeros((L,), jnp.float32)
        return carry
    lax.fori_loop(0, CH, _zrow, 0)
    _zero_acc()
    plsc.subcore_barrier()

    for l in range(N_LAYERS):
        src = x0s_hbm.at[c] if l == 0 else xs_hbm.at[c, l - 1]

        _eb_issue(0, 0)
        _eb_issue(1, 1)
        _eb_wait(0, 0)
        _eb_wait(1, 1)
        for b in range(NBUF):
            pltpu.async_copy(src.at[colb.at[0, b]], gbuf.at[b], gsem.at[b])

        def _group(g, carry):
            u = lax.rem(g, EB)
            un = lax.rem(g + 1, EB)
            uf = lax.rem(g + 2, EB)
            for b in range(NBUF):
                sb = b % 2
                pltpu.make_async_copy(src.at[colb.at[u, b]], gbuf.at[b],
                                      gsem.at[b]).wait()
                if b >= 2:
                    pltpu.make_async_copy(
                        sbuf.at[sb], acc_sh.at[rowb.at[u, b - 2]],
                        ssem.at[sb]).wait()
                else:
                    @pl.when(g > 0)
                    def _():
                        pltpu.make_async_copy(
                            sbuf.at[sb], acc_sh.at[rowb.at[uf, b + 2]],
                            ssem.at[sb]).wait()

                def _scale(hh, carry2):
                    vv = valb[u, b, pl.ds(hh * L, L)]
                    for t in range(L):
                        e = hh * L + t
                        v = vv[t]
                        for q in range(D // L):
                            sbuf[sb, e, pl.ds(q * L, L)] = \
                                gbuf[b, e, pl.ds(q * L, L)] * v
                    return carry2
                lax.fori_loop(0, CH // L, _scale, 0)
                pltpu.async_copy(sbuf.at[sb], acc_sh.at[rowb.at[u, b]],
                                 ssem.at[sb], add=True)
                @pl.when(g < NGRP - 1)
                def _():
                    pltpu.async_copy(src.at[colb.at[un, b]], gbuf.at[b],
                                     gsem.at[b])
                if b == 1:
                    @pl.when(g < NGRP - 2)
                    def _():
                        _eb_issue(g + 2, uf)

            @pl.when(g < NGRP - 2)
            def _():
                _eb_wait(g + 2, uf)
            return carry
        lax.fori_loop(0, NGRP, _group, 0)

        ul = (NGRP - 1) % EB
        pltpu.make_async_copy(sbuf.at[0], acc_sh.at[rowb.at[ul, 2]],
                              ssem.at[0]).wait()
        pltpu.make_async_copy(sbuf.at[1], acc_sh.at[rowb.at[ul, 3]],
                              ssem.at[1]).wait()
        plsc.subcore_barrier()

        sl = pl.ds(s * ROWS_PER_SUB, ROWS_PER_SUB)
        pltpu.async_copy(acc_sh.at[sl], xs_hbm.at[c, l, sl], wsem)
        pltpu.make_async_copy(acc_sh.at[sl], xs_hbm.at[c, l, sl], wsem).wait()
        if l < N_LAYERS - 1:
            _zero_acc()
        plsc.subcore_barrier()


@functools.partial(
    pl.kernel,
    out_type=jax.ShapeDtypeStruct((BATCH,), jnp.float32),
    mesh=_mesh,
    scratch_types=[
        pltpu.VMEM((BPW,), jnp.int32),
        pltpu.VMEM((BPW,), jnp.int32),
        pltpu.VMEM((BPW,), jnp.int32),
        pltpu.VMEM((NUM_ITEMS,), jnp.int32),
        pltpu.VMEM((BPW, D), jnp.float32),
        pltpu.VMEM((BPW, D), jnp.float32),
        pltpu.VMEM((BPW, D), jnp.float32),
        pltpu.VMEM((BPW,), jnp.float32),
        pltpu.SemaphoreType.DMA,
    ],
    compiler_params=_params,
)
def _readout(users_hbm, items_hbm, map_hbm, x0s_hbm, xs_hbm, gamma_hbm,
             uidx, iidx, midx, map_v, uacc, iacc, tmp, gout, sem):
    c = lax.axis_index("c")
    s = lax.axis_index("s")
    wid = s * NC + c
    base = wid * BPW
    pltpu.sync_copy(users_hbm.at[pl.ds(base, BPW)], uidx)
    pltpu.sync_copy(items_hbm.at[pl.ds(base, BPW)], iidx)
    pltpu.sync_copy(map_hbm, map_v)

    for g in range(BPW // L):
        ivec = iidx[pl.ds(g * L, L)]
        m = plsc.load_gather(map_v, [ivec])
        midx[pl.ds(g * L, L)] = m + NUM_USERS
        iidx[pl.ds(g * L, L)] = ivec + NUM_USERS

    tables = [x0s_hbm.at[0], x0s_hbm.at[1]] + \
        [xs_hbm.at[cc, ll] for ll in range(N_LAYERS) for cc in range(NC)]

    def _acc_into(accbuf, first_tbl, rest):
        pltpu.async_copy(first_tbl, accbuf, sem).wait()
        for tbl in rest:
            pltpu.async_copy(tbl, tmp, sem).wait()

            def _addrow(e, carry):
                for q in range(D // L):
                    sl = pl.ds(q * L, L)
                    accbuf[e, sl] = accbuf[e, sl] + tmp[e, sl]
                return carry
            lax.fori_loop(0, BPW, _addrow, 0)

    _acc_into(uacc, tables[0].at[uidx], [t.at[uidx] for t in tables[1:]])
    _acc_into(iacc, tables[0].at[iidx],
              [tables[k].at[iidx] for k in (2, 4, 6)] +
              [tables[k].at[midx] for k in (1, 3, 5, 7)])

    lane = lax.broadcasted_iota(jnp.int32, (L,), 0)

    def _dot(g, carry):
        gvec = jnp.zeros((L,), jnp.float32)
        for t in range(L):
            e = g * L + t
            p = uacc[e, pl.ds(0, L)] * iacc[e, pl.ds(0, L)]
            for q in range(1, D // L):
                sl = pl.ds(q * L, L)
                p = p + uacc[e, sl] * iacc[e, sl]
            gvec = jnp.where(lane == t, jnp.sum(p) * (1.0 / 64.0), gvec)
        gout[pl.ds(g * L, L)] = gvec
        return carry
    lax.fori_loop(0, BPW // L, _dot, 0)
    pltpu.sync_copy(gout, gamma_hbm.at[pl.ds(base, BPW)])


def _pad_edges(a):
    return jnp.pad(a, (0, E_PAD - NNZ)).reshape(NS, NCHUNK, CH)


def kernel(users, items, user_emb_0, user_emb_1, item_emb_0, item_emb_1,
           edge_row_0, edge_col_0, edge_val_0,
           edge_row_1, edge_col_1, edge_val_1, map_list):
    x0s = jnp.stack([jnp.concatenate([user_emb_0, item_emb_0], axis=0),
                     jnp.concatenate([user_emb_1, item_emb_1], axis=0)])
    colp = jnp.stack([_pad_edges(edge_col_0), _pad_edges(edge_col_1)])
    rowp = jnp.stack([_pad_edges(edge_row_0), _pad_edges(edge_row_1)])
    valp = jnp.stack([_pad_edges(edge_val_0), _pad_edges(edge_val_1)])
    xs = _propagate(x0s, colp, rowp, valp)
    return _readout(users, items, map_list, x0s, xs)


# speedup vs baseline: 15.4050x; 1.0028x over previous
import functools

import jax
import jax.numpy as jnp
from jax import lax
from jax.experimental import pallas as pl
from jax.experimental.pallas import tpu as pltpu
from jax.experimental.pallas import tpu_sc as plsc

NUM_USERS = 8192
NUM_ITEMS = 8192
N = NUM_USERS + NUM_ITEMS
NNZ = 268435
D = 64
BATCH = 4096
N_LAYERS = 3

NC = 2
NS = 16
L = 16

CH = 128
E_PER_SUB = -(-NNZ // NS)
NCHUNK = -(-E_PER_SUB // CH)
E_SUB_PAD = NCHUNK * CH
E_PAD = E_SUB_PAD * NS

ROWS_PER_SUB = N // NS
BPW = BATCH // (NC * NS)

NBUF = 4
NGRP = NCHUNK // NBUF
EB = 3
assert NCHUNK % NBUF == 0

_mesh = plsc.VectorSubcoreMesh(core_axis_name="c", subcore_axis_name="s")
_params = pltpu.CompilerParams(use_tc_tiling_on_sc=False,
                               needs_layout_passes=False)


@functools.partial(
    pl.kernel,
    out_type=jax.ShapeDtypeStruct((NC, N_LAYERS, N, D), jnp.float32),
    mesh=_mesh,
    scratch_types=[
        pltpu.VMEM((EB, NBUF, CH), jnp.int32),
        pltpu.VMEM((EB, NBUF, CH), jnp.int32),
        pltpu.VMEM((EB, NBUF, CH), jnp.float32),
        pltpu.VMEM((NBUF, CH, D), jnp.float32),
        pltpu.VMEM((2, CH, D), jnp.float32),
        pltpu.VMEM((CH, D), jnp.float32),
        pltpu.VMEM_SHARED((N, D), jnp.float32),
        pltpu.SemaphoreType.DMA((EB,)),
        pltpu.SemaphoreType.DMA((NBUF,)),
        pltpu.SemaphoreType.DMA((2,)),
        pltpu.SemaphoreType.DMA,
    ],
    compiler_params=_params,
)
def _propagate(x0s_hbm, col_hbm, row_hbm, val_hbm, xs_hbm,
               colb, rowb, valb, gbuf, sbuf, zbuf, acc_sh,
               esem, gsem, ssem, wsem):
    c = lax.axis_index("c")
    s = lax.axis_index("s")

    def _eb_issue(g, u):
        pltpu.async_copy(col_hbm.at[c, s, pl.ds(g * NBUF, NBUF)],
                         colb.at[u], esem.at[u])
        pltpu.async_copy(row_hbm.at[c, s, pl.ds(g * NBUF, NBUF)],
                         rowb.at[u], esem.at[u])
        pltpu.async_copy(val_hbm.at[c, s, pl.ds(g * NBUF, NBUF)],
                         valb.at[u], esem.at[u])

    def _eb_wait(g, u):
        pltpu.make_async_copy(col_hbm.at[c, s, pl.ds(g * NBUF, NBUF)],
                              colb.at[u], esem.at[u]).wait()
        pltpu.make_async_copy(row_hbm.at[c, s, pl.ds(g * NBUF, NBUF)],
                              rowb.at[u], esem.at[u]).wait()
        pltpu.make_async_copy(val_hbm.at[c, s, pl.ds(g * NBUF, NBUF)],
                              valb.at[u], esem.at[u]).wait()

    def _zero_acc():
        for i in range(ROWS_PER_SUB // CH):
            pltpu.async_copy(
                zbuf, acc_sh.at[pl.ds(s * ROWS_PER_SUB + i * CH, CH)], wsem)
        for i in range(ROWS_PER_SUB // CH):
            pltpu.make_async_copy(
                zbuf, acc_sh.at[pl.ds(s * ROWS_PER_SUB + i * CH, CH)],
                wsem).wait()

    def _zrow(e, carry):
        for q in range(D // L):
            zbuf[e, pl.ds(q * L, L)] = jnp.zeros((L,), jnp.float32)
        return carry
    lax.fori_loop(0, CH, _zrow, 0)
    _zero_acc()
    plsc.subcore_barrier()

    for l in range(N_LAYERS):
        src = x0s_hbm.at[c] if l == 0 else xs_hbm.at[c, l - 1]

        _eb_issue(0, 0)
        _eb_issue(1, 1)
        _eb_wait(0, 0)
        _eb_wait(1, 1)
        for b in range(NBUF):
            pltpu.async_copy(src.at[colb.at[0, b]], gbuf.at[b], gsem.at[b])

        def _group(g, carry):
            u = lax.rem(g, EB)
            un = lax.rem(g + 1, EB)
            uf = lax.rem(g + 2, EB)
            for b in range(NBUF):
                sb = b % 2
                pltpu.make_async_copy(src.at[colb.at[u, b]], gbuf.at[b],
                                      gsem.at[b]).wait()
                if b >= 2:
                    pltpu.make_async_copy(
                        sbuf.at[sb], acc_sh.at[rowb.at[u, b - 2]],
                        ssem.at[sb]).wait()
                else:
                    @pl.when(g > 0)
                    def _():
                        pltpu.make_async_copy(
                            sbuf.at[sb], acc_sh.at[rowb.at[uf, b + 2]],
                            ssem.at[sb]).wait()

                @plsc.parallel_loop(0, CH, step=L, unroll=2)
                def _scale(e0):
                    vv = valb[u, b, pl.ds(e0, L)]
                    for t in range(L):
                        v = vv[t]
                        for q in range(D // L):
                            sbuf[sb, e0 + t, pl.ds(q * L, L)] = \
                                gbuf[b, e0 + t, pl.ds(q * L, L)] * v
                pltpu.async_copy(sbuf.at[sb], acc_sh.at[rowb.at[u, b]],
                                 ssem.at[sb], add=True)
                @pl.when(g < NGRP - 1)
                def _():
                    pltpu.async_copy(src.at[colb.at[un, b]], gbuf.at[b],
                                     gsem.at[b])
                if b == 1:
                    @pl.when(g < NGRP - 2)
                    def _():
                        _eb_issue(g + 2, uf)

            @pl.when(g < NGRP - 2)
            def _():
                _eb_wait(g + 2, uf)
            return carry
        lax.fori_loop(0, NGRP, _group, 0)

        ul = (NGRP - 1) % EB
        pltpu.make_async_copy(sbuf.at[0], acc_sh.at[rowb.at[ul, 2]],
                              ssem.at[0]).wait()
        pltpu.make_async_copy(sbuf.at[1], acc_sh.at[rowb.at[ul, 3]],
                              ssem.at[1]).wait()
        plsc.subcore_barrier()

        sl = pl.ds(s * ROWS_PER_SUB, ROWS_PER_SUB)
        pltpu.async_copy(acc_sh.at[sl], xs_hbm.at[c, l, sl], wsem)
        pltpu.make_async_copy(acc_sh.at[sl], xs_hbm.at[c, l, sl], wsem).wait()
        if l < N_LAYERS - 1:
            _zero_acc()
        plsc.subcore_barrier()


@functools.partial(
    pl.kernel,
    out_type=jax.ShapeDtypeStruct((BATCH,), jnp.float32),
    mesh=_mesh,
    scratch_types=[
        pltpu.VMEM((BPW,), jnp.int32),
        pltpu.VMEM((BPW,), jnp.int32),
        pltpu.VMEM((BPW,), jnp.int32),
        pltpu.VMEM((NUM_ITEMS,), jnp.int32),
        pltpu.VMEM((BPW, D), jnp.float32),
        pltpu.VMEM((BPW, D), jnp.float32),
        pltpu.VMEM((BPW, D), jnp.float32),
        pltpu.VMEM((BPW,), jnp.float32),
        pltpu.SemaphoreType.DMA,
    ],
    compiler_params=_params,
)
def _readout(users_hbm, items_hbm, map_hbm, x0s_hbm, xs_hbm, gamma_hbm,
             uidx, iidx, midx, map_v, uacc, iacc, tmp, gout, sem):
    c = lax.axis_index("c")
    s = lax.axis_index("s")
    wid = s * NC + c
    base = wid * BPW
    pltpu.sync_copy(users_hbm.at[pl.ds(base, BPW)], uidx)
    pltpu.sync_copy(items_hbm.at[pl.ds(base, BPW)], iidx)
    pltpu.sync_copy(map_hbm, map_v)

    for g in range(BPW // L):
        ivec = iidx[pl.ds(g * L, L)]
        m = plsc.load_gather(map_v, [ivec])
        midx[pl.ds(g * L, L)] = m + NUM_USERS
        iidx[pl.ds(g * L, L)] = ivec + NUM_USERS

    tables = [x0s_hbm.at[0], x0s_hbm.at[1]] + \
        [xs_hbm.at[cc, ll] for ll in range(N_LAYERS) for cc in range(NC)]

    def _acc_into(accbuf, first_tbl, rest):
        pltpu.async_copy(first_tbl, accbuf, sem).wait()
        for tbl in rest:
            pltpu.async_copy(tbl, tmp, sem).wait()

            def _addrow(e, carry):
                for q in range(D // L):
                    sl = pl.ds(q * L, L)
                    accbuf[e, sl] = accbuf[e, sl] + tmp[e, sl]
                return carry
            lax.fori_loop(0, BPW, _addrow, 0)

    _acc_into(uacc, tables[0].at[uidx], [t.at[uidx] for t in tables[1:]])
    _acc_into(iacc, tables[0].at[iidx],
              [tables[k].at[iidx] for k in (2, 4, 6)] +
              [tables[k].at[midx] for k in (1, 3, 5, 7)])

    lane = lax.broadcasted_iota(jnp.int32, (L,), 0)

    def _dot(g, carry):
        gvec = jnp.zeros((L,), jnp.float32)
        for t in range(L):
            e = g * L + t
            p = uacc[e, pl.ds(0, L)] * iacc[e, pl.ds(0, L)]
            for q in range(1, D // L):
                sl = pl.ds(q * L, L)
                p = p + uacc[e, sl] * iacc[e, sl]
            gvec = jnp.where(lane == t, jnp.sum(p) * (1.0 / 64.0), gvec)
        gout[pl.ds(g * L, L)] = gvec
        return carry
    lax.fori_loop(0, BPW // L, _dot, 0)
    pltpu.sync_copy(gout, gamma_hbm.at[pl.ds(base, BPW)])


def _pad_edges(a):
    return jnp.pad(a, (0, E_PAD - NNZ)).reshape(NS, NCHUNK, CH)


def kernel(users, items, user_emb_0, user_emb_1, item_emb_0, item_emb_1,
           edge_row_0, edge_col_0, edge_val_0,
           edge_row_1, edge_col_1, edge_val_1, map_list):
    x0s = jnp.stack([jnp.concatenate([user_emb_0, item_emb_0], axis=0),
                     jnp.concatenate([user_emb_1, item_emb_1], axis=0)])
    colp = jnp.stack([_pad_edges(edge_col_0), _pad_edges(edge_col_1)])
    rowp = jnp.stack([_pad_edges(edge_row_0), _pad_edges(edge_row_1)])
    valp = jnp.stack([_pad_edges(edge_val_0), _pad_edges(edge_val_1)])
    xs = _propagate(x0s, colp, rowp, valp)
    return _readout(users, items, map_list, x0s, xs)
